# baseline (speedup 1.0000x reference)
import functools

import jax
import jax.numpy as jnp
from jax import lax
from jax.experimental import pallas as pl
from jax.experimental.pallas import tpu as pltpu
from jax.experimental.pallas import tpu_sc as plsc

EMBED = 64
HIDDEN = 64

NC = 2
NS = 16
NW = NC * NS

CHUNK = 128


def _sc_gather(table, idx3, tok, nchunk, b, l):
    per_w = nchunk * CHUNK
    half = b // 2
    mesh = plsc.VectorSubcoreMesh(core_axis_name="c", subcore_axis_name="s")

    @functools.partial(
        pl.kernel,
        out_type=jax.ShapeDtypeStruct((tok // 2, 2 * EMBED), jnp.float32),
        mesh=mesh,
        scratch_types=[
            pltpu.VMEM((nchunk, CHUNK), jnp.int32),
            pltpu.VMEM((2, CHUNK, EMBED), jnp.float32),
            pltpu.SemaphoreType.DMA,
            pltpu.SemaphoreType.DMA,
        ],
        compiler_params=pltpu.CompilerParams(use_tc_tiling_on_sc=False),
    )
    def k(table_hbm, idx_hbm, out_hbm, idx_v, rows_v, gsem0, gsem1):
        wid = lax.axis_index("s") * NC + lax.axis_index("c")
        base = wid * per_w
        pltpu.sync_copy(idx_hbm.at[wid], idx_v)

        gsems = (gsem0, gsem1)
        for buf in range(2):
            pltpu.async_copy(
                table_hbm.at[idx_v.at[buf]], rows_v.at[buf], gsems[buf]
            )

        def handle(j, buf):
            pltpu.make_async_copy(
                table_hbm.at[idx_v.at[j]], rows_v.at[buf], gsems[buf]
            ).wait()
            s = base + j * CHUNK
            li = s // b
            r = s - li * b
            p = r // half
            bp = r - p * half
            pltpu.sync_copy(
                rows_v.at[buf],
                out_hbm.at[
                    pl.ds(li * half + bp, CHUNK), pl.ds(p * EMBED, EMBED)
                ],
            )

            @pl.when(j + 2 < nchunk)
            def _():
                pltpu.async_copy(
                    table_hbm.at[idx_v.at[j + 2]], rows_v.at[buf], gsems[buf]
                )

        def body(jj, _):
            handle(2 * jj, 0)
            handle(2 * jj + 1, 1)
            return 0

        lax.fori_loop(0, nchunk // 2, body, 0)

    return k(table, idx3)


VBLK = 2000


def _tc_pack_table(table, v):
    hv = VBLK // 2
    grid = v // VBLK

    def body(t_ref, out_ref):
        out_ref[:, :EMBED] = t_ref[:hv, :]
        out_ref[:, EMBED:] = t_ref[hv:, :]

    return pl.pallas_call(
        body,
        grid=(grid,),
        in_specs=[pl.BlockSpec((VBLK, EMBED), lambda i: (i, 0))],
        out_specs=pl.BlockSpec((hv, 2 * EMBED), lambda i: (i, 0)),
        out_shape=jax.ShapeDtypeStruct((v // 2, 2 * EMBED), jnp.float32),
    )(table)


def _tc_project(emb128, w2d, b, l):
    half = b // 2

    def body(emb_ref, w_ref, out_ref):
        h = lax.dot_general(
            w_ref[...],
            emb_ref[...],
            dimension_numbers=(((1,), (1,)), ((), ())),
            preferred_element_type=jnp.float32,
        )
        inv_sqrt2 = 0.70710678118654752
        g = 0.5 * h * (1.0 + lax.erf(h * inv_sqrt2))
        out_ref[:, :half] = g[:HIDDEN, :]
        out_ref[:, half:] = g[HIDDEN:, :]

    return pl.pallas_call(
        body,
        grid=(l,),
        in_specs=[
            pl.BlockSpec((half, 2 * EMBED), lambda li: (li, 0)),
            pl.BlockSpec((2 * EMBED, 2 * EMBED), lambda li: (0, 0)),
        ],
        out_specs=pl.BlockSpec((HIDDEN, b), lambda li: (li, 0)),
        out_shape=jax.ShapeDtypeStruct((l * HIDDEN, b), jnp.float32),
    )(emb128, w2d)


def kernel(x, table, W):
    b, l = x.shape
    tok = b * l
    nchunk = tok // (NW * CHUNK)

    v = table.shape[0]
    hv = VBLK // 2
    xt = x.T.astype(jnp.int32)
    u = xt % VBLK
    xr = (xt - u) + 2 * (u % hv) + u // hv
    idx3 = xr.reshape(NW, nchunk, CHUNK)

    table_p = _tc_pack_table(table, v).reshape(v, EMBED)
    emb128 = _sc_gather(table_p, idx3, tok, nchunk, b, l)

    w2d = jnp.zeros((2 * EMBED, 2 * EMBED), jnp.float32)
    w2d = w2d.at[:HIDDEN, :EMBED].set(W).at[HIDDEN:, EMBED:].set(W)

    out2d = _tc_project(emb128, w2d, b, l)
    return out2d.reshape(l, HIDDEN, b).transpose(2, 0, 1)


# speedup vs baseline: 1.2160x; 1.2160x over previous
import functools

import jax
import jax.numpy as jnp
from jax import lax
from jax.experimental import pallas as pl
from jax.experimental.pallas import tpu as pltpu
from jax.experimental.pallas import tpu_sc as plsc

EMBED = 64
HIDDEN = 64

NC = 2
NS = 16
NW = NC * NS

CHUNK = 128


def _sc_gather(table, idx3, tok, nchunk, b, l):
    per_w = nchunk * CHUNK
    half = b // 2
    mesh = plsc.VectorSubcoreMesh(core_axis_name="c", subcore_axis_name="s")

    @functools.partial(
        pl.kernel,
        out_type=jax.ShapeDtypeStruct((tok // 2, 2 * EMBED), jnp.float32),
        mesh=mesh,
        scratch_types=[
            pltpu.VMEM((nchunk, CHUNK), jnp.int32),
            pltpu.VMEM((2, CHUNK, EMBED), jnp.float32),
            pltpu.SemaphoreType.DMA,
            pltpu.SemaphoreType.DMA,
        ],
        compiler_params=pltpu.CompilerParams(use_tc_tiling_on_sc=False),
    )
    def k(table_hbm, idx_hbm, out_hbm, idx_v, rows_v, gsem0, gsem1):
        wid = lax.axis_index("s") * NC + lax.axis_index("c")
        base = wid * per_w
        pltpu.sync_copy(idx_hbm.at[wid], idx_v)

        gsems = (gsem0, gsem1)
        for buf in range(2):
            pltpu.async_copy(
                table_hbm.at[idx_v.at[buf]], rows_v.at[buf], gsems[buf]
            )

        def handle(j, buf):
            pltpu.make_async_copy(
                table_hbm.at[idx_v.at[j]], rows_v.at[buf], gsems[buf]
            ).wait()
            s = base + j * CHUNK
            li = s // b
            r = s - li * b
            p = r // half
            bp = r - p * half
            pltpu.sync_copy(
                rows_v.at[buf],
                out_hbm.at[
                    pl.ds(li * half + bp, CHUNK), pl.ds(p * EMBED, EMBED)
                ],
            )

            @pl.when(j + 2 < nchunk)
            def _():
                pltpu.async_copy(
                    table_hbm.at[idx_v.at[j + 2]], rows_v.at[buf], gsems[buf]
                )

        def body(jj, _):
            handle(2 * jj, 0)
            handle(2 * jj + 1, 1)
            return 0

        lax.fori_loop(0, nchunk // 2, body, 0)

    return k(table, idx3)


VBLK = 2000


def _tc_pack_table(table, v):
    hv = VBLK // 2
    grid = v // VBLK

    def body(t_ref, out_ref):
        out_ref[:, :EMBED] = t_ref[:hv, :]
        out_ref[:, EMBED:] = t_ref[hv:, :]

    return pl.pallas_call(
        body,
        grid=(grid,),
        in_specs=[pl.BlockSpec((VBLK, EMBED), lambda i: (i, 0))],
        out_specs=pl.BlockSpec((hv, 2 * EMBED), lambda i: (i, 0)),
        out_shape=jax.ShapeDtypeStruct((v // 2, 2 * EMBED), jnp.float32),
    )(table)


def _tc_project(emb128, w2d, b, l):
    half = b // 2

    def body(emb_ref, w_ref, out_ref):
        h = lax.dot_general(
            w_ref[...],
            emb_ref[...],
            dimension_numbers=(((1,), (1,)), ((), ())),
            preferred_element_type=jnp.float32,
        )
        inv_sqrt2 = 0.70710678118654752
        g = 0.5 * h * (1.0 + lax.erf(h * inv_sqrt2))
        out_ref[:, :half] = g[:HIDDEN, :]
        out_ref[:, half:] = g[HIDDEN:, :]

    return pl.pallas_call(
        body,
        grid=(l,),
        in_specs=[
            pl.BlockSpec((half, 2 * EMBED), lambda li: (li, 0)),
            pl.BlockSpec((2 * EMBED, 2 * EMBED), lambda li: (0, 0)),
        ],
        out_specs=pl.BlockSpec((HIDDEN, b), lambda li: (li, 0)),
        out_shape=jax.ShapeDtypeStruct((l * HIDDEN, b), jnp.float32),
    )(emb128, w2d)


def _tc_project_chunk(emb128, w2d, prev, l0, nl, ltot, b):
    half = b // 2

    def body(emb_ref, w_ref, *rest):
        out_ref = rest[-1]
        h = lax.dot_general(
            w_ref[...],
            emb_ref[...],
            dimension_numbers=(((1,), (1,)), ((), ())),
            preferred_element_type=jnp.float32,
        )
        inv_sqrt2 = 0.70710678118654752
        g = 0.5 * h * (1.0 + lax.erf(h * inv_sqrt2))
        out_ref[:, :half] = g[:HIDDEN, :]
        out_ref[:, half:] = g[HIDDEN:, :]

    in_specs = [
        pl.BlockSpec((half, 2 * EMBED), lambda li: (li, 0)),
        pl.BlockSpec((2 * EMBED, 2 * EMBED), lambda li: (0, 0)),
    ]
    args = [emb128, w2d]
    aliases = {}
    if prev is not None:
        in_specs.append(pl.BlockSpec((8, 128), lambda li: (0, 0)))
        args.append(prev)
        aliases = {2: 0}
    return pl.pallas_call(
        body,
        grid=(nl,),
        in_specs=in_specs,
        out_specs=pl.BlockSpec((HIDDEN, b), lambda li: (li + l0, 0)),
        out_shape=jax.ShapeDtypeStruct((ltot * HIDDEN, b), jnp.float32),
        input_output_aliases=aliases,
    )(*args)


def kernel(x, table, W):
    b, l = x.shape
    tok = b * l
    nchunk = tok // (NW * CHUNK)

    xt = x.T.astype(jnp.int32)
    ha = l // 2

    w2d = jnp.zeros((2 * EMBED, 2 * EMBED), jnp.float32)
    w2d = w2d.at[:HIDDEN, :EMBED].set(W).at[HIDDEN:, EMBED:].set(W)

    idx3a = xt[:ha].reshape(NW, (ha * b) // (NW * CHUNK), CHUNK)
    idx3b = xt[ha:].reshape(NW, ((l - ha) * b) // (NW * CHUNK), CHUNK)
    emb_a = _sc_gather(table, idx3a, ha * b, (ha * b) // (NW * CHUNK), b, ha)
    emb_b = _sc_gather(
        table, idx3b, (l - ha) * b, ((l - ha) * b) // (NW * CHUNK), b, l - ha
    )
    out1 = _tc_project_chunk(emb_a, w2d, None, 0, ha, l, b)
    out2d = _tc_project_chunk(emb_b, w2d, out1, ha, l - ha, l, b)
    return out2d.reshape(l, HIDDEN, b).transpose(2, 0, 1)


# speedup vs baseline: 1.2166x; 1.0005x over previous
import functools

import jax
import jax.numpy as jnp
from jax import lax
from jax.experimental import pallas as pl
from jax.experimental.pallas import tpu as pltpu
from jax.experimental.pallas import tpu_sc as plsc

EMBED = 64
HIDDEN = 64

NC = 2
NS = 16
NW = NC * NS

CHUNK = 128


def _sc_gather(table, idx3, tok, nchunk, b, l):
    per_w = nchunk * CHUNK
    half = b // 2
    mesh = plsc.VectorSubcoreMesh(core_axis_name="c", subcore_axis_name="s")

    @functools.partial(
        pl.kernel,
        out_type=jax.ShapeDtypeStruct((tok // 2, 2 * EMBED), jnp.float32),
        mesh=mesh,
        scratch_types=[
            pltpu.VMEM((nchunk, CHUNK), jnp.int32),
            pltpu.VMEM((2, CHUNK, EMBED), jnp.float32),
            pltpu.SemaphoreType.DMA,
            pltpu.SemaphoreType.DMA,
        ],
        compiler_params=pltpu.CompilerParams(use_tc_tiling_on_sc=False),
    )
    def k(table_hbm, idx_hbm, out_hbm, idx_v, rows_v, gsem0, gsem1):
        wid = lax.axis_index("s") * NC + lax.axis_index("c")
        base = wid * per_w
        pltpu.sync_copy(idx_hbm.at[wid], idx_v)

        gsems = (gsem0, gsem1)
        for buf in range(2):
            pltpu.async_copy(
                table_hbm.at[idx_v.at[buf]], rows_v.at[buf], gsems[buf]
            )

        def handle(j, buf):
            pltpu.make_async_copy(
                table_hbm.at[idx_v.at[j]], rows_v.at[buf], gsems[buf]
            ).wait()
            s = base + j * CHUNK
            li = s // b
            r = s - li * b
            p = r // half
            bp = r - p * half
            pltpu.sync_copy(
                rows_v.at[buf],
                out_hbm.at[
                    pl.ds(li * half + bp, CHUNK), pl.ds(p * EMBED, EMBED)
                ],
            )

            @pl.when(j + 2 < nchunk)
            def _():
                pltpu.async_copy(
                    table_hbm.at[idx_v.at[j + 2]], rows_v.at[buf], gsems[buf]
                )

        def body(jj, _):
            handle(2 * jj, 0)
            handle(2 * jj + 1, 1)
            return 0

        lax.fori_loop(0, nchunk // 2, body, 0)

    return k(table, idx3)


def _tc_project_chunk(emb128, w2d, prev, l0, nl, ltot, b):
    half = b // 2

    def body(emb_ref, w_ref, *rest):
        out_ref = rest[-1]
        h = lax.dot_general(
            w_ref[...],
            emb_ref[...],
            dimension_numbers=(((1,), (1,)), ((), ())),
            preferred_element_type=jnp.float32,
        )
        inv_sqrt2 = 0.70710678118654752
        g = 0.5 * h * (1.0 + lax.erf(h * inv_sqrt2))
        out_ref[:, :half] = g[:HIDDEN, :]
        out_ref[:, half:] = g[HIDDEN:, :]

    in_specs = [
        pl.BlockSpec((half, 2 * EMBED), lambda li: (li, 0)),
        pl.BlockSpec((2 * EMBED, 2 * EMBED), lambda li: (0, 0)),
    ]
    args = [emb128, w2d]
    aliases = {}
    if prev is not None:
        in_specs.append(pl.BlockSpec((8, 128), lambda li: (0, 0)))
        args.append(prev)
        aliases = {2: 0}
    return pl.pallas_call(
        body,
        grid=(nl,),
        in_specs=in_specs,
        out_specs=pl.BlockSpec((HIDDEN, b), lambda li: (li + l0, 0)),
        out_shape=jax.ShapeDtypeStruct((ltot * HIDDEN, b), jnp.float32),
        input_output_aliases=aliases,
    )(*args)


def kernel(x, table, W):
    b, l = x.shape
    tok = b * l
    nchunk = tok // (NW * CHUNK)

    xt = x.T.astype(jnp.int32)
    ha = l // 2

    w2d = jnp.zeros((2 * EMBED, 2 * EMBED), jnp.float32)
    w2d = w2d.at[:HIDDEN, :EMBED].set(W).at[HIDDEN:, EMBED:].set(W)

    idx3a = xt[:ha].reshape(NW, (ha * b) // (NW * CHUNK), CHUNK)
    idx3b = xt[ha:].reshape(NW, ((l - ha) * b) // (NW * CHUNK), CHUNK)
    emb_a = _sc_gather(table, idx3a, ha * b, (ha * b) // (NW * CHUNK), b, ha)
    emb_b = _sc_gather(
        table, idx3b, (l - ha) * b, ((l - ha) * b) // (NW * CHUNK), b, l - ha
    )
    out1 = _tc_project_chunk(emb_a, w2d, None, 0, ha, l, b)
    out2d = _tc_project_chunk(emb_b, w2d, out1, ha, l - ha, l, b)
    return out2d.reshape(l, HIDDEN, b).transpose(2, 0, 1)
